# baseline (speedup 1.0000x reference)
import jax
import jax.numpy as jnp
from jax import lax
from jax.experimental import pallas as pl
from jax.experimental.pallas import tpu as pltpu
from jax.experimental.pallas import tpu_sc as plsc

_N = 10000
_DH = 128
_NC = 2
_NS = 16
_NW = _NC * _NS
_CHUNK = 128
_CHUNKS = 80
_KBUF = 2
_GROUPS = _CHUNKS // _KBUF
_PW = _CHUNK * _CHUNKS
_E_PAD = _PW * _NW
_N_ACC = 10112
_RPS = _N_ACC // _NS
_BM = 2000


def _make_seg_sum(d, with_cnt):
  mesh = plsc.VectorSubcoreMesh(core_axis_name="c", subcore_axis_name="s")
  out_type = (jax.ShapeDtypeStruct((_NC, _N_ACC, d), jnp.float32),)
  if with_cnt:
    out_type += (jax.ShapeDtypeStruct((_NC * _N_ACC,), jnp.float32),)
  scratch = (
      [pltpu.VMEM((_PW,), jnp.int32)]
      + [pltpu.VMEM((_CHUNK,), jnp.int32) for _ in range(_KBUF)]
      + [pltpu.VMEM((_CHUNK, d), jnp.float32) for _ in range(_KBUF)]
      + [
          pltpu.VMEM((_CHUNK,), jnp.float32),
          pltpu.VMEM((_RPS,), jnp.float32),
          pltpu.VMEM_SHARED((_N_ACC, d), jnp.float32),
          pltpu.VMEM_SHARED((_N_ACC,), jnp.float32),
      ]
      + [pltpu.SemaphoreType.DMA for _ in range(4 * _KBUF)]
  )

  def body(p_hbm, src_hbm, dst_hbm, z2_hbm, z1_hbm, *refs):
    if with_cnt:
      part_out, cnt_out = refs[0], refs[1]
      rest = refs[2:]
    else:
      part_out = refs[0]
      cnt_out = None
      rest = refs[1:]
    src_all = rest[0]
    dst_v = rest[1:1 + _KBUF]
    rows_v = rest[1 + _KBUF:1 + 2 * _KBUF]
    ones_v, stage_v, acc, cnta = rest[1 + 2 * _KBUF:1 + 2 * _KBUF + 4]
    sems = rest[1 + 2 * _KBUF + 4:]
    sem_g = sems[:_KBUF]
    sem_i = sems[_KBUF:2 * _KBUF]
    sem_s = sems[2 * _KBUF:3 * _KBUF]
    sem_o = sems[3 * _KBUF:4 * _KBUF]
    c = lax.axis_index("c")
    s = lax.axis_index("s")
    w = s * _NC + c
    wb = w * _PW
    pltpu.sync_copy(src_hbm.at[pl.ds(wb, _PW)], src_all)
    pltpu.sync_copy(z2_hbm.at[pl.ds(s * _RPS, _RPS)],
                    acc.at[pl.ds(s * _RPS, _RPS)])
    if with_cnt:
      pltpu.sync_copy(z1_hbm.at[pl.ds(s * _RPS, _RPS)], stage_v)
      pltpu.sync_copy(stage_v, cnta.at[pl.ds(s * _RPS, _RPS)])
      for i in range(_CHUNK // 16):
        ones_v[pl.ds(i * 16, 16)] = jnp.full((16,), 1.0, jnp.float32)
    plsc.subcore_barrier()

    def issue_fetch(cidx, i):
      pltpu.async_copy(dst_hbm.at[pl.ds(wb + cidx * _CHUNK, _CHUNK)],
                       dst_v[i], sem_i[i])
      pltpu.async_copy(p_hbm.at[src_all.at[pl.ds(cidx * _CHUNK, _CHUNK)]],
                       rows_v[i], sem_g[i])

    def wait_fetch(cidx, i):
      pltpu.make_async_copy(dst_hbm.at[pl.ds(wb + cidx * _CHUNK, _CHUNK)],
                            dst_v[i], sem_i[i]).wait()
      pltpu.make_async_copy(
          p_hbm.at[src_all.at[pl.ds(cidx * _CHUNK, _CHUNK)]],
          rows_v[i], sem_g[i]).wait()

    def issue_scatter(i):
      pltpu.async_copy(rows_v[i], acc.at[dst_v[i]], sem_s[i], add=True)
      if with_cnt:
        pltpu.async_copy(ones_v, cnta.at[dst_v[i]], sem_o[i], add=True)

    def wait_scatter(i):
      pltpu.make_async_copy(rows_v[i], acc.at[dst_v[i]], sem_s[i]).wait()
      if with_cnt:
        pltpu.make_async_copy(ones_v, cnta.at[dst_v[i]], sem_o[i]).wait()

    for i in range(_KBUF):
      issue_fetch(i, i)

    def grp(g, carry):
      j = g * _KBUF
      for i in range(_KBUF):
        wait_fetch(j + i, i)
        issue_scatter(i)

      @pl.when(g < _GROUPS - 1)
      def _prefetch():
        for i in range(_KBUF):
          wait_scatter(i)
          issue_fetch(j + _KBUF + i, i)

      return carry

    lax.fori_loop(0, _GROUPS, grp, 0)
    for i in range(_KBUF):
      wait_scatter(i)
    plsc.subcore_barrier()
    pltpu.sync_copy(acc.at[pl.ds(s * _RPS, _RPS)],
                    part_out.at[c, pl.ds(s * _RPS, _RPS)])
    if with_cnt:
      pltpu.sync_copy(cnta.at[pl.ds(s * _RPS, _RPS)], stage_v)
      pltpu.sync_copy(stage_v,
                      cnt_out.at[pl.ds(c * _N_ACC + s * _RPS, _RPS)])

  return pl.kernel(body, out_type=out_type, scratch_types=scratch, mesh=mesh)


_C3 = 64
_CH3 = _PW // _C3
_K3 = 4
_RPS2 = 2 * _N_ACC // _NS


def _make_seg_sum2():
  mesh = plsc.VectorSubcoreMesh(core_axis_name="c", subcore_axis_name="s")
  out_type = jax.ShapeDtypeStruct((_NC * 2 * _N_ACC,), jnp.float32)
  scratch = (
      [
          pltpu.VMEM((2 * _N,), jnp.float32),
          pltpu.VMEM((_PW,), jnp.int32),
          pltpu.VMEM((_PW,), jnp.int32),
          pltpu.VMEM((_RPS2,), jnp.float32),
      ]
      + [pltpu.VMEM((2 * _C3,), jnp.float32) for _ in range(_K3)]
      + [pltpu.VMEM((2 * _C3,), jnp.int32) for _ in range(_K3)]
      + [pltpu.VMEM_SHARED((2 * _N_ACC,), jnp.float32)]
      + [pltpu.SemaphoreType.DMA for _ in range(_K3)]
  )

  def body(p_hbm, src_hbm, dst_hbm, out_ref, *refs):
    p_loc, src_all, dst_all, stage_v = refs[:4]
    vals_v = refs[4:4 + _K3]
    eidx_v = refs[4 + _K3:4 + 2 * _K3]
    acc = refs[4 + 2 * _K3]
    sem_s = refs[4 + 2 * _K3 + 1:]
    c = lax.axis_index("c")
    s = lax.axis_index("s")
    w = s * _NC + c
    wb = w * _PW
    pltpu.sync_copy(p_hbm, p_loc)
    pltpu.sync_copy(src_hbm.at[pl.ds(wb, _PW)], src_all)
    pltpu.sync_copy(dst_hbm.at[pl.ds(wb, _PW)], dst_all)
    for i in range(_RPS2 // 16):
      stage_v[pl.ds(i * 16, 16)] = jnp.zeros((16,), jnp.float32)
    pltpu.sync_copy(stage_v, acc.at[pl.ds(s * _RPS2, _RPS2)])
    plsc.subcore_barrier()

    iot = lax.iota(jnp.int32, 16)
    half = lax.shift_right_logical(iot, 1)
    par = lax.bitwise_and(iot, 1)

    def fill(cidx, b):
      for k in range(_C3 // 8):
        off = cidx * _C3 + k * 8
        pos = off + half
        s16 = plsc.load_gather(src_all, [pos])
        vals = plsc.load_gather(p_loc, [s16 * 2 + par])
        vals_v[b][pl.ds(k * 16, 16)] = vals
        d16 = plsc.load_gather(dst_all, [pos])
        eidx_v[b][pl.ds(k * 16, 16)] = d16 * 2 + par

    def grp(g, carry):
      for i in range(_K3):
        j = g * _K3 + i

        @pl.when(g > 0)
        def _wait():
          pltpu.make_async_copy(vals_v[i], acc.at[eidx_v[i]],
                                sem_s[i]).wait()

        fill(j, i)
        pltpu.async_copy(vals_v[i], acc.at[eidx_v[i]], sem_s[i], add=True)
      return carry

    lax.fori_loop(0, _CH3 // _K3, grp, 0)
    for i in range(_K3):
      pltpu.make_async_copy(vals_v[i], acc.at[eidx_v[i]], sem_s[i]).wait()
    plsc.subcore_barrier()
    pltpu.sync_copy(acc.at[pl.ds(s * _RPS2, _RPS2)], stage_v)
    pltpu.sync_copy(stage_v,
                    out_ref.at[pl.ds(c * 2 * _N_ACC + s * _RPS2, _RPS2)])

  return pl.kernel(
      body, out_type=out_type, scratch_types=scratch, mesh=mesh,
      compiler_params=pltpu.CompilerParams(needs_layout_passes=False))


def _mm_body(x_ref, w_ref, o_ref):
  o_ref[...] = jnp.dot(x_ref[...], w_ref[...],
                       preferred_element_type=jnp.float32)


def _matmul(x, W):
  n, k = x.shape
  m = W.shape[1]
  return pl.pallas_call(
      _mm_body,
      grid=(n // _BM,),
      in_specs=[pl.BlockSpec((_BM, k), lambda i: (i, 0)),
                pl.BlockSpec((k, m), lambda i: (0, 0))],
      out_specs=pl.BlockSpec((_BM, m), lambda i: (i, 0)),
      out_shape=jax.ShapeDtypeStruct((n, m), jnp.float32),
  )(x, W)


def _stage_body(h_ref, parts_ref, cnt_ref, wr_ref, b_ref, g_ref, be_ref,
                wn_ref, h_out, p_out):
  cnt = cnt_ref[:, 0:1] + cnt_ref[:, 1:2]
  inv = 1.0 / jnp.maximum(cnt, 1.0)
  m = (parts_ref[0] + parts_ref[1]) * inv
  z = m + jnp.dot(h_ref[...], wr_ref[...],
                  preferred_element_type=jnp.float32) + b_ref[...]
  mu = jnp.mean(z, axis=-1, keepdims=True)
  var = jnp.mean((z - mu) ** 2, axis=-1, keepdims=True)
  y = (z - mu) * lax.rsqrt(var + 1e-5) * g_ref[...] + be_ref[...]
  h = jnp.maximum(y, 0.0)
  h_out[...] = h
  p_out[...] = jnp.dot(h, wn_ref[...], preferred_element_type=jnp.float32)


def _stage(h, parts, cntT, Wr, b, g, be, Wn):
  dn = Wn.shape[1]
  return pl.pallas_call(
      _stage_body,
      grid=(_N // _BM,),
      in_specs=[
          pl.BlockSpec((_BM, _DH), lambda i: (i, 0)),
          pl.BlockSpec((_NC, _BM, _DH), lambda i: (0, i, 0)),
          pl.BlockSpec((_BM, _NC), lambda i: (i, 0)),
          pl.BlockSpec((_DH, _DH), lambda i: (0, 0)),
          pl.BlockSpec((1, _DH), lambda i: (0, 0)),
          pl.BlockSpec((1, _DH), lambda i: (0, 0)),
          pl.BlockSpec((1, _DH), lambda i: (0, 0)),
          pl.BlockSpec((_DH, dn), lambda i: (0, 0)),
      ],
      out_specs=[pl.BlockSpec((_BM, _DH), lambda i: (i, 0)),
                 pl.BlockSpec((_BM, dn), lambda i: (i, 0))],
      out_shape=[jax.ShapeDtypeStruct((_N, _DH), jnp.float32),
                 jax.ShapeDtypeStruct((_N, dn), jnp.float32)],
  )(h, parts, cntT, Wr, b, g, be, Wn)


def _final_body(h_ref, parts_ref, cnt_ref, wr_ref, b_ref, o_ref):
  cnt = cnt_ref[:, 0:1] + cnt_ref[:, 1:2]
  inv = 1.0 / jnp.maximum(cnt, 1.0)
  m = (parts_ref[0] + parts_ref[1]) * inv
  o_ref[...] = m + jnp.dot(h_ref[...], wr_ref[...],
                           preferred_element_type=jnp.float32) + b_ref[...]


def _final(h, parts, cntT, Wr, b):
  dn = Wr.shape[1]
  return pl.pallas_call(
      _final_body,
      grid=(_N // _BM,),
      in_specs=[
          pl.BlockSpec((_BM, _DH), lambda i: (i, 0)),
          pl.BlockSpec((_NC, _BM, dn), lambda i: (0, i, 0)),
          pl.BlockSpec((_BM, _NC), lambda i: (i, 0)),
          pl.BlockSpec((_DH, dn), lambda i: (0, 0)),
          pl.BlockSpec((1, dn), lambda i: (0, 0)),
      ],
      out_specs=pl.BlockSpec((_BM, dn), lambda i: (i, 0)),
      out_shape=jax.ShapeDtypeStruct((_N, dn), jnp.float32),
  )(h, parts, cntT, Wr, b)


def kernel(x, edge_index, Wl1, Wr1, b1, g1, be1, Wl2, Wr2, b2, g2, be2,
           Wl3, Wr3, b3):
  src = edge_index[0]
  dst = edge_index[1]
  pad = _E_PAD - src.shape[0]
  ar = jnp.arange(pad, dtype=jnp.int32)
  src_p = jnp.concatenate([src, ar % _N])
  dst_p = jnp.concatenate([dst, _N + (ar % 16)])
  z2 = jnp.zeros((_N_ACC, _DH), jnp.float32)
  z1 = jnp.zeros((_N_ACC,), jnp.float32)

  seg_cnt = _make_seg_sum(_DH, True)
  seg128 = _make_seg_sum(_DH, False)

  b1r, g1r, be1r = b1.reshape(1, -1), g1.reshape(1, -1), be1.reshape(1, -1)
  b2r, g2r, be2r = b2.reshape(1, -1), g2.reshape(1, -1), be2.reshape(1, -1)
  b3r = b3.reshape(1, -1)

  p1 = _matmul(x, Wl1)
  parts1, cnt = seg_cnt(p1, src_p, dst_p, z2, z1)
  cntT = jnp.transpose(cnt.reshape(_NC, _N_ACC))
  h1, p2 = _stage(x, parts1, cntT, Wr1, b1r, g1r, be1r, Wl2)
  (parts2,) = seg128(p2, src_p, dst_p, z2, z1)
  h2, p3 = _stage(h1, parts2, cntT, Wr2, b2r, g2r, be2r, Wl3)
  seg2 = _make_seg_sum2()
  parts3 = seg2(p3.reshape(2 * _N), src_p, dst_p).reshape(_NC, _N_ACC, 2)
  return _final(h2, parts3, cntT, Wr3, b3r)


# speedup vs baseline: 1.1468x; 1.1468x over previous
import jax
import jax.numpy as jnp
from jax import lax
from jax.experimental import pallas as pl
from jax.experimental.pallas import tpu as pltpu
from jax.experimental.pallas import tpu_sc as plsc

_N = 10000
_DH = 128
_NC = 2
_NS = 16
_NW = _NC * _NS
_CHUNK = 48
_CHUNKS = 216
_KBUF = 6
_GROUPS = _CHUNKS // _KBUF
_PW = _CHUNK * _CHUNKS
_E_PAD = _PW * _NW
_N_ACC = 10112
_RPS = _N_ACC // _NS
_BM = 2000


def _make_seg_sum(d, with_cnt):
  mesh = plsc.VectorSubcoreMesh(core_axis_name="c", subcore_axis_name="s")
  out_type = (jax.ShapeDtypeStruct((_NC, _N_ACC, d), jnp.float32),)
  if with_cnt:
    out_type += (jax.ShapeDtypeStruct((_NC * _N_ACC,), jnp.float32),)
  scratch = (
      [pltpu.VMEM((_PW,), jnp.int32)]
      + [pltpu.VMEM((_CHUNK,), jnp.int32) for _ in range(_KBUF)]
      + [pltpu.VMEM((_CHUNK, d), jnp.float32) for _ in range(_KBUF)]
      + [
          pltpu.VMEM((_CHUNK,), jnp.float32),
          pltpu.VMEM((_RPS,), jnp.float32),
          pltpu.VMEM_SHARED((_N_ACC, d), jnp.float32),
          pltpu.VMEM_SHARED((_N_ACC,), jnp.float32),
      ]
      + [pltpu.SemaphoreType.DMA for _ in range(4 * _KBUF)]
  )

  def body(p_hbm, src_hbm, dst_hbm, z2_hbm, z1_hbm, *refs):
    if with_cnt:
      part_out, cnt_out = refs[0], refs[1]
      rest = refs[2:]
    else:
      part_out = refs[0]
      cnt_out = None
      rest = refs[1:]
    src_all = rest[0]
    dst_v = rest[1:1 + _KBUF]
    rows_v = rest[1 + _KBUF:1 + 2 * _KBUF]
    ones_v, stage_v, acc, cnta = rest[1 + 2 * _KBUF:1 + 2 * _KBUF + 4]
    sems = rest[1 + 2 * _KBUF + 4:]
    sem_g = sems[:_KBUF]
    sem_i = sems[_KBUF:2 * _KBUF]
    sem_s = sems[2 * _KBUF:3 * _KBUF]
    sem_o = sems[3 * _KBUF:4 * _KBUF]
    c = lax.axis_index("c")
    s = lax.axis_index("s")
    w = s * _NC + c
    wb = w * _PW
    pltpu.sync_copy(src_hbm.at[pl.ds(wb, _PW)], src_all)
    pltpu.sync_copy(z2_hbm.at[pl.ds(s * _RPS, _RPS)],
                    acc.at[pl.ds(s * _RPS, _RPS)])
    if with_cnt:
      pltpu.sync_copy(z1_hbm.at[pl.ds(s * _RPS, _RPS)], stage_v)
      pltpu.sync_copy(stage_v, cnta.at[pl.ds(s * _RPS, _RPS)])
      for i in range(_CHUNK // 16):
        ones_v[pl.ds(i * 16, 16)] = jnp.full((16,), 1.0, jnp.float32)
    plsc.subcore_barrier()

    def issue_fetch(cidx, i):
      pltpu.async_copy(dst_hbm.at[pl.ds(wb + cidx * _CHUNK, _CHUNK)],
                       dst_v[i], sem_i[i])
      pltpu.async_copy(p_hbm.at[src_all.at[pl.ds(cidx * _CHUNK, _CHUNK)]],
                       rows_v[i], sem_g[i])

    def wait_fetch(cidx, i):
      pltpu.make_async_copy(dst_hbm.at[pl.ds(wb + cidx * _CHUNK, _CHUNK)],
                            dst_v[i], sem_i[i]).wait()
      pltpu.make_async_copy(
          p_hbm.at[src_all.at[pl.ds(cidx * _CHUNK, _CHUNK)]],
          rows_v[i], sem_g[i]).wait()

    def issue_scatter(i):
      pltpu.async_copy(rows_v[i], acc.at[dst_v[i]], sem_s[i], add=True)
      if with_cnt:
        pltpu.async_copy(ones_v, cnta.at[dst_v[i]], sem_o[i], add=True)

    def wait_scatter(i):
      pltpu.make_async_copy(rows_v[i], acc.at[dst_v[i]], sem_s[i]).wait()
      if with_cnt:
        pltpu.make_async_copy(ones_v, cnta.at[dst_v[i]], sem_o[i]).wait()

    for i in range(_KBUF):
      issue_fetch(i, i)

    def grp(g, carry):
      j = g * _KBUF
      for i in range(_KBUF):
        wait_fetch(j + i, i)
        issue_scatter(i)

      @pl.when(g < _GROUPS - 1)
      def _prefetch():
        for i in range(_KBUF):
          wait_scatter(i)
          issue_fetch(j + _KBUF + i, i)

      return carry

    lax.fori_loop(0, _GROUPS, grp, 0)
    for i in range(_KBUF):
      wait_scatter(i)
    plsc.subcore_barrier()
    pltpu.sync_copy(acc.at[pl.ds(s * _RPS, _RPS)],
                    part_out.at[c, pl.ds(s * _RPS, _RPS)])
    if with_cnt:
      pltpu.sync_copy(cnta.at[pl.ds(s * _RPS, _RPS)], stage_v)
      pltpu.sync_copy(stage_v,
                      cnt_out.at[pl.ds(c * _N_ACC + s * _RPS, _RPS)])

  return pl.kernel(body, out_type=out_type, scratch_types=scratch, mesh=mesh)


_C3 = 32
_CH3 = _PW // _C3
_K3 = 4
_RPS2 = 2 * _N_ACC // _NS


def _make_seg_sum2():
  mesh = plsc.VectorSubcoreMesh(core_axis_name="c", subcore_axis_name="s")
  out_type = jax.ShapeDtypeStruct((_NC * 2 * _N_ACC,), jnp.float32)
  scratch = (
      [
          pltpu.VMEM((2 * _N,), jnp.float32),
          pltpu.VMEM((_PW,), jnp.int32),
          pltpu.VMEM((_PW,), jnp.int32),
          pltpu.VMEM((_RPS2,), jnp.float32),
      ]
      + [pltpu.VMEM((2 * _C3,), jnp.float32) for _ in range(_K3)]
      + [pltpu.VMEM((2 * _C3,), jnp.int32) for _ in range(_K3)]
      + [pltpu.VMEM_SHARED((2 * _N_ACC,), jnp.float32)]
      + [pltpu.SemaphoreType.DMA for _ in range(_K3)]
  )

  def body(p_hbm, src_hbm, dst_hbm, out_ref, *refs):
    p_loc, src_all, dst_all, stage_v = refs[:4]
    vals_v = refs[4:4 + _K3]
    eidx_v = refs[4 + _K3:4 + 2 * _K3]
    acc = refs[4 + 2 * _K3]
    sem_s = refs[4 + 2 * _K3 + 1:]
    c = lax.axis_index("c")
    s = lax.axis_index("s")
    w = s * _NC + c
    wb = w * _PW
    pltpu.sync_copy(p_hbm, p_loc)
    pltpu.sync_copy(src_hbm.at[pl.ds(wb, _PW)], src_all)
    pltpu.sync_copy(dst_hbm.at[pl.ds(wb, _PW)], dst_all)
    for i in range(_RPS2 // 16):
      stage_v[pl.ds(i * 16, 16)] = jnp.zeros((16,), jnp.float32)
    pltpu.sync_copy(stage_v, acc.at[pl.ds(s * _RPS2, _RPS2)])
    plsc.subcore_barrier()

    iot = lax.iota(jnp.int32, 16)
    half = lax.shift_right_logical(iot, 1)
    par = lax.bitwise_and(iot, 1)

    def fill(cidx, b):
      for k in range(_C3 // 8):
        off = cidx * _C3 + k * 8
        pos = off + half
        s16 = plsc.load_gather(src_all, [pos])
        vals = plsc.load_gather(p_loc, [s16 * 2 + par])
        vals_v[b][pl.ds(k * 16, 16)] = vals
        d16 = plsc.load_gather(dst_all, [pos])
        eidx_v[b][pl.ds(k * 16, 16)] = d16 * 2 + par

    def grp(g, carry):
      for i in range(_K3):
        j = g * _K3 + i

        @pl.when(g > 0)
        def _wait():
          pltpu.make_async_copy(vals_v[i], acc.at[eidx_v[i]],
                                sem_s[i]).wait()

        fill(j, i)
        pltpu.async_copy(vals_v[i], acc.at[eidx_v[i]], sem_s[i], add=True)
      return carry

    lax.fori_loop(0, _CH3 // _K3, grp, 0)
    for i in range(_K3):
      pltpu.make_async_copy(vals_v[i], acc.at[eidx_v[i]], sem_s[i]).wait()
    plsc.subcore_barrier()
    pltpu.sync_copy(acc.at[pl.ds(s * _RPS2, _RPS2)], stage_v)
    pltpu.sync_copy(stage_v,
                    out_ref.at[pl.ds(c * 2 * _N_ACC + s * _RPS2, _RPS2)])

  return pl.kernel(
      body, out_type=out_type, scratch_types=scratch, mesh=mesh,
      compiler_params=pltpu.CompilerParams(needs_layout_passes=False))


def _mm_body(x_ref, w_ref, o_ref):
  o_ref[...] = jnp.dot(x_ref[...], w_ref[...],
                       preferred_element_type=jnp.float32)


def _matmul(x, W):
  n, k = x.shape
  m = W.shape[1]
  return pl.pallas_call(
      _mm_body,
      grid=(n // _BM,),
      in_specs=[pl.BlockSpec((_BM, k), lambda i: (i, 0)),
                pl.BlockSpec((k, m), lambda i: (0, 0))],
      out_specs=pl.BlockSpec((_BM, m), lambda i: (i, 0)),
      out_shape=jax.ShapeDtypeStruct((n, m), jnp.float32),
  )(x, W)


def _stage_body(h_ref, parts_ref, cnt_ref, wr_ref, b_ref, g_ref, be_ref,
                wn_ref, h_out, p_out):
  cnt = cnt_ref[:, 0:1] + cnt_ref[:, 1:2]
  inv = 1.0 / jnp.maximum(cnt, 1.0)
  m = (parts_ref[0] + parts_ref[1]) * inv
  z = m + jnp.dot(h_ref[...], wr_ref[...],
                  preferred_element_type=jnp.float32) + b_ref[...]
  mu = jnp.mean(z, axis=-1, keepdims=True)
  var = jnp.mean((z - mu) ** 2, axis=-1, keepdims=True)
  y = (z - mu) * lax.rsqrt(var + 1e-5) * g_ref[...] + be_ref[...]
  h = jnp.maximum(y, 0.0)
  h_out[...] = h
  p_out[...] = jnp.dot(h, wn_ref[...], preferred_element_type=jnp.float32)


def _stage(h, parts, cntT, Wr, b, g, be, Wn):
  dn = Wn.shape[1]
  return pl.pallas_call(
      _stage_body,
      grid=(_N // _BM,),
      in_specs=[
          pl.BlockSpec((_BM, _DH), lambda i: (i, 0)),
          pl.BlockSpec((_NC, _BM, _DH), lambda i: (0, i, 0)),
          pl.BlockSpec((_BM, _NC), lambda i: (i, 0)),
          pl.BlockSpec((_DH, _DH), lambda i: (0, 0)),
          pl.BlockSpec((1, _DH), lambda i: (0, 0)),
          pl.BlockSpec((1, _DH), lambda i: (0, 0)),
          pl.BlockSpec((1, _DH), lambda i: (0, 0)),
          pl.BlockSpec((_DH, dn), lambda i: (0, 0)),
      ],
      out_specs=[pl.BlockSpec((_BM, _DH), lambda i: (i, 0)),
                 pl.BlockSpec((_BM, dn), lambda i: (i, 0))],
      out_shape=[jax.ShapeDtypeStruct((_N, _DH), jnp.float32),
                 jax.ShapeDtypeStruct((_N, dn), jnp.float32)],
  )(h, parts, cntT, Wr, b, g, be, Wn)


def _final_body(h_ref, parts_ref, cnt_ref, wr_ref, b_ref, o_ref):
  cnt = cnt_ref[:, 0:1] + cnt_ref[:, 1:2]
  inv = 1.0 / jnp.maximum(cnt, 1.0)
  m = (parts_ref[0] + parts_ref[1]) * inv
  o_ref[...] = m + jnp.dot(h_ref[...], wr_ref[...],
                           preferred_element_type=jnp.float32) + b_ref[...]


def _final(h, parts, cntT, Wr, b):
  dn = Wr.shape[1]
  return pl.pallas_call(
      _final_body,
      grid=(_N // _BM,),
      in_specs=[
          pl.BlockSpec((_BM, _DH), lambda i: (i, 0)),
          pl.BlockSpec((_NC, _BM, dn), lambda i: (0, i, 0)),
          pl.BlockSpec((_BM, _NC), lambda i: (i, 0)),
          pl.BlockSpec((_DH, dn), lambda i: (0, 0)),
          pl.BlockSpec((1, dn), lambda i: (0, 0)),
      ],
      out_specs=pl.BlockSpec((_BM, dn), lambda i: (i, 0)),
      out_shape=jax.ShapeDtypeStruct((_N, dn), jnp.float32),
  )(h, parts, cntT, Wr, b)


def kernel(x, edge_index, Wl1, Wr1, b1, g1, be1, Wl2, Wr2, b2, g2, be2,
           Wl3, Wr3, b3):
  src = edge_index[0]
  dst = edge_index[1]
  pad = _E_PAD - src.shape[0]
  ar = jnp.arange(pad, dtype=jnp.int32)
  src_p = jnp.concatenate([src, ar % _N])
  dst_p = jnp.concatenate([dst, _N + (ar % 16)])
  z2 = jnp.zeros((_N_ACC, _DH), jnp.float32)
  z1 = jnp.zeros((_N_ACC,), jnp.float32)

  seg_cnt = _make_seg_sum(_DH, True)
  seg128 = _make_seg_sum(_DH, False)

  b1r, g1r, be1r = b1.reshape(1, -1), g1.reshape(1, -1), be1.reshape(1, -1)
  b2r, g2r, be2r = b2.reshape(1, -1), g2.reshape(1, -1), be2.reshape(1, -1)
  b3r = b3.reshape(1, -1)

  p1 = _matmul(x, Wl1)
  parts1, cnt = seg_cnt(p1, src_p, dst_p, z2, z1)
  cntT = jnp.transpose(cnt.reshape(_NC, _N_ACC))
  h1, p2 = _stage(x, parts1, cntT, Wr1, b1r, g1r, be1r, Wl2)
  (parts2,) = seg128(p2, src_p, dst_p, z2, z1)
  h2, p3 = _stage(h1, parts2, cntT, Wr2, b2r, g2r, be2r, Wl3)
  seg2 = _make_seg_sum2()
  parts3 = seg2(p3.reshape(2 * _N), src_p, dst_p).reshape(_NC, _N_ACC, 2)
  return _final(h2, parts3, cntT, Wr3, b3r)


# speedup vs baseline: 1.1623x; 1.0136x over previous
import jax
import jax.numpy as jnp
from jax import lax
from jax.experimental import pallas as pl
from jax.experimental.pallas import tpu as pltpu
from jax.experimental.pallas import tpu_sc as plsc

_N = 10000
_DH = 128
_NC = 2
_NS = 16
_NW = _NC * _NS
_CHUNK = 48
_CHUNKS = 216
_KBUF = 6
_GROUPS = _CHUNKS // _KBUF
_PW = _CHUNK * _CHUNKS
_E_PAD = _PW * _NW
_N_ACC = 10112
_RPS = _N_ACC // _NS
_BM = 2000


def _make_seg_sum(d, with_cnt):
  mesh = plsc.VectorSubcoreMesh(core_axis_name="c", subcore_axis_name="s")
  out_type = (jax.ShapeDtypeStruct((_NC, _N_ACC, d), jnp.float32),)
  if with_cnt:
    out_type += (jax.ShapeDtypeStruct((_NC * _N_ACC,), jnp.float32),)
  scratch = (
      [pltpu.VMEM((_PW,), jnp.int32)]
      + [pltpu.VMEM((_CHUNK,), jnp.int32) for _ in range(_KBUF)]
      + [pltpu.VMEM((_CHUNK, d), jnp.float32) for _ in range(_KBUF)]
      + [
          pltpu.VMEM((_CHUNK,), jnp.float32),
          pltpu.VMEM((_RPS,), jnp.float32),
          pltpu.VMEM_SHARED((_N_ACC, d), jnp.float32),
          pltpu.VMEM_SHARED((_N_ACC,), jnp.float32),
      ]
      + [pltpu.SemaphoreType.DMA for _ in range(4 * _KBUF)]
  )

  def body(p_hbm, src_hbm, dst_hbm, z2_hbm, z1_hbm, *refs):
    if with_cnt:
      part_out, cnt_out = refs[0], refs[1]
      rest = refs[2:]
    else:
      part_out = refs[0]
      cnt_out = None
      rest = refs[1:]
    src_all = rest[0]
    dst_v = rest[1:1 + _KBUF]
    rows_v = rest[1 + _KBUF:1 + 2 * _KBUF]
    ones_v, stage_v, acc, cnta = rest[1 + 2 * _KBUF:1 + 2 * _KBUF + 4]
    sems = rest[1 + 2 * _KBUF + 4:]
    sem_g = sems[:_KBUF]
    sem_i = sems[_KBUF:2 * _KBUF]
    sem_s = sems[2 * _KBUF:3 * _KBUF]
    sem_o = sems[3 * _KBUF:4 * _KBUF]
    c = lax.axis_index("c")
    s = lax.axis_index("s")
    w = s * _NC + c
    wb = w * _PW
    pltpu.sync_copy(src_hbm.at[pl.ds(wb, _PW)], src_all)
    def issue_fetch(cidx, i):
      pltpu.async_copy(dst_hbm.at[pl.ds(wb + cidx * _CHUNK, _CHUNK)],
                       dst_v[i], sem_i[i])
      pltpu.async_copy(p_hbm.at[src_all.at[pl.ds(cidx * _CHUNK, _CHUNK)]],
                       rows_v[i], sem_g[i])

    def wait_fetch(cidx, i):
      pltpu.make_async_copy(dst_hbm.at[pl.ds(wb + cidx * _CHUNK, _CHUNK)],
                            dst_v[i], sem_i[i]).wait()
      pltpu.make_async_copy(
          p_hbm.at[src_all.at[pl.ds(cidx * _CHUNK, _CHUNK)]],
          rows_v[i], sem_g[i]).wait()

    def issue_scatter(i):
      pltpu.async_copy(rows_v[i], acc.at[dst_v[i]], sem_s[i], add=True)
      if with_cnt:
        pltpu.async_copy(ones_v, cnta.at[dst_v[i]], sem_o[i], add=True)

    def wait_scatter(i):
      pltpu.make_async_copy(rows_v[i], acc.at[dst_v[i]], sem_s[i]).wait()
      if with_cnt:
        pltpu.make_async_copy(ones_v, cnta.at[dst_v[i]], sem_o[i]).wait()

    for i in range(_KBUF):
      issue_fetch(i, i)
    pltpu.sync_copy(z2_hbm.at[pl.ds(s * _RPS, _RPS)],
                    acc.at[pl.ds(s * _RPS, _RPS)])
    if with_cnt:
      pltpu.sync_copy(z1_hbm.at[pl.ds(s * _RPS, _RPS)], stage_v)
      pltpu.sync_copy(stage_v, cnta.at[pl.ds(s * _RPS, _RPS)])
      for i in range(_CHUNK // 16):
        ones_v[pl.ds(i * 16, 16)] = jnp.full((16,), 1.0, jnp.float32)
    plsc.subcore_barrier()

    def grp(g, carry):
      j = g * _KBUF
      for i in range(_KBUF):
        wait_fetch(j + i, i)
        issue_scatter(i)

      @pl.when(g < _GROUPS - 1)
      def _prefetch():
        for i in range(_KBUF):
          wait_scatter(i)
          issue_fetch(j + _KBUF + i, i)

      return carry

    lax.fori_loop(0, _GROUPS, grp, 0)
    for i in range(_KBUF):
      wait_scatter(i)
    plsc.subcore_barrier()
    pltpu.sync_copy(acc.at[pl.ds(s * _RPS, _RPS)],
                    part_out.at[c, pl.ds(s * _RPS, _RPS)])
    if with_cnt:
      pltpu.sync_copy(cnta.at[pl.ds(s * _RPS, _RPS)], stage_v)
      pltpu.sync_copy(stage_v,
                      cnt_out.at[pl.ds(c * _N_ACC + s * _RPS, _RPS)])

  return pl.kernel(body, out_type=out_type, scratch_types=scratch, mesh=mesh)


_C3 = 32
_CH3 = _PW // _C3
_K3 = 4
_RPS2 = 2 * _N_ACC // _NS


def _make_seg_sum2():
  mesh = plsc.VectorSubcoreMesh(core_axis_name="c", subcore_axis_name="s")
  out_type = jax.ShapeDtypeStruct((_NC * 2 * _N_ACC,), jnp.float32)
  scratch = (
      [
          pltpu.VMEM((2 * _N,), jnp.float32),
          pltpu.VMEM((_PW,), jnp.int32),
          pltpu.VMEM((_PW,), jnp.int32),
          pltpu.VMEM((_RPS2,), jnp.float32),
      ]
      + [pltpu.VMEM((2 * _C3,), jnp.float32) for _ in range(_K3)]
      + [pltpu.VMEM((2 * _C3,), jnp.int32) for _ in range(_K3)]
      + [pltpu.VMEM_SHARED((2 * _N_ACC,), jnp.float32)]
      + [pltpu.SemaphoreType.DMA for _ in range(_K3 + 3)]
  )

  def body(p_hbm, src_hbm, dst_hbm, out_ref, *refs):
    p_loc, src_all, dst_all, stage_v = refs[:4]
    vals_v = refs[4:4 + _K3]
    eidx_v = refs[4 + _K3:4 + 2 * _K3]
    acc = refs[4 + 2 * _K3]
    sem_s = refs[4 + 2 * _K3 + 1:4 + 3 * _K3 + 1]
    sem_in = refs[4 + 3 * _K3 + 1:]
    c = lax.axis_index("c")
    s = lax.axis_index("s")
    w = s * _NC + c
    wb = w * _PW
    pltpu.async_copy(p_hbm, p_loc, sem_in[0])
    pltpu.async_copy(src_hbm.at[pl.ds(wb, _PW)], src_all, sem_in[1])
    pltpu.async_copy(dst_hbm.at[pl.ds(wb, _PW)], dst_all, sem_in[2])
    for i in range(_RPS2 // 16):
      stage_v[pl.ds(i * 16, 16)] = jnp.zeros((16,), jnp.float32)
    pltpu.sync_copy(stage_v, acc.at[pl.ds(s * _RPS2, _RPS2)])
    pltpu.make_async_copy(p_hbm, p_loc, sem_in[0]).wait()
    pltpu.make_async_copy(src_hbm.at[pl.ds(wb, _PW)], src_all,
                          sem_in[1]).wait()
    pltpu.make_async_copy(dst_hbm.at[pl.ds(wb, _PW)], dst_all,
                          sem_in[2]).wait()
    plsc.subcore_barrier()

    iot = lax.iota(jnp.int32, 16)
    half = lax.shift_right_logical(iot, 1)
    par = lax.bitwise_and(iot, 1)

    def fill(cidx, b):
      for k in range(_C3 // 8):
        off = cidx * _C3 + k * 8
        pos = off + half
        s16 = plsc.load_gather(src_all, [pos])
        vals = plsc.load_gather(p_loc, [s16 * 2 + par])
        vals_v[b][pl.ds(k * 16, 16)] = vals
        d16 = plsc.load_gather(dst_all, [pos])
        eidx_v[b][pl.ds(k * 16, 16)] = d16 * 2 + par

    def grp(g, carry):
      for i in range(_K3):
        j = g * _K3 + i

        @pl.when(g > 0)
        def _wait():
          pltpu.make_async_copy(vals_v[i], acc.at[eidx_v[i]],
                                sem_s[i]).wait()

        fill(j, i)
        pltpu.async_copy(vals_v[i], acc.at[eidx_v[i]], sem_s[i], add=True)
      return carry

    lax.fori_loop(0, _CH3 // _K3, grp, 0)
    for i in range(_K3):
      pltpu.make_async_copy(vals_v[i], acc.at[eidx_v[i]], sem_s[i]).wait()
    plsc.subcore_barrier()
    pltpu.sync_copy(acc.at[pl.ds(s * _RPS2, _RPS2)], stage_v)
    pltpu.sync_copy(stage_v,
                    out_ref.at[pl.ds(c * 2 * _N_ACC + s * _RPS2, _RPS2)])

  return pl.kernel(
      body, out_type=out_type, scratch_types=scratch, mesh=mesh,
      compiler_params=pltpu.CompilerParams(needs_layout_passes=False))


def _mm_body(x_ref, w_ref, o_ref):
  o_ref[...] = jnp.dot(x_ref[...], w_ref[...],
                       preferred_element_type=jnp.float32)


def _matmul(x, W):
  n, k = x.shape
  m = W.shape[1]
  return pl.pallas_call(
      _mm_body,
      grid=(n // _BM,),
      in_specs=[pl.BlockSpec((_BM, k), lambda i: (i, 0)),
                pl.BlockSpec((k, m), lambda i: (0, 0))],
      out_specs=pl.BlockSpec((_BM, m), lambda i: (i, 0)),
      out_shape=jax.ShapeDtypeStruct((n, m), jnp.float32),
  )(x, W)


def _stage_body(h_ref, parts_ref, cnt_ref, wr_ref, b_ref, g_ref, be_ref,
                wn_ref, h_out, p_out):
  cnt = cnt_ref[:, 0:1] + cnt_ref[:, 1:2]
  inv = 1.0 / jnp.maximum(cnt, 1.0)
  m = (parts_ref[0] + parts_ref[1]) * inv
  z = m + jnp.dot(h_ref[...], wr_ref[...],
                  preferred_element_type=jnp.float32) + b_ref[...]
  mu = jnp.mean(z, axis=-1, keepdims=True)
  var = jnp.mean((z - mu) ** 2, axis=-1, keepdims=True)
  y = (z - mu) * lax.rsqrt(var + 1e-5) * g_ref[...] + be_ref[...]
  h = jnp.maximum(y, 0.0)
  h_out[...] = h
  p_out[...] = jnp.dot(h, wn_ref[...], preferred_element_type=jnp.float32)


def _stage(h, parts, cntT, Wr, b, g, be, Wn):
  dn = Wn.shape[1]
  return pl.pallas_call(
      _stage_body,
      grid=(_N // _BM,),
      in_specs=[
          pl.BlockSpec((_BM, _DH), lambda i: (i, 0)),
          pl.BlockSpec((_NC, _BM, _DH), lambda i: (0, i, 0)),
          pl.BlockSpec((_BM, _NC), lambda i: (i, 0)),
          pl.BlockSpec((_DH, _DH), lambda i: (0, 0)),
          pl.BlockSpec((1, _DH), lambda i: (0, 0)),
          pl.BlockSpec((1, _DH), lambda i: (0, 0)),
          pl.BlockSpec((1, _DH), lambda i: (0, 0)),
          pl.BlockSpec((_DH, dn), lambda i: (0, 0)),
      ],
      out_specs=[pl.BlockSpec((_BM, _DH), lambda i: (i, 0)),
                 pl.BlockSpec((_BM, dn), lambda i: (i, 0))],
      out_shape=[jax.ShapeDtypeStruct((_N, _DH), jnp.float32),
                 jax.ShapeDtypeStruct((_N, dn), jnp.float32)],
  )(h, parts, cntT, Wr, b, g, be, Wn)


def _final_body(h_ref, parts_ref, cnt_ref, wr_ref, b_ref, o_ref):
  cnt = cnt_ref[:, 0:1] + cnt_ref[:, 1:2]
  inv = 1.0 / jnp.maximum(cnt, 1.0)
  m = (parts_ref[0] + parts_ref[1]) * inv
  o_ref[...] = m + jnp.dot(h_ref[...], wr_ref[...],
                           preferred_element_type=jnp.float32) + b_ref[...]


def _final(h, parts, cntT, Wr, b):
  dn = Wr.shape[1]
  return pl.pallas_call(
      _final_body,
      grid=(_N // _BM,),
      in_specs=[
          pl.BlockSpec((_BM, _DH), lambda i: (i, 0)),
          pl.BlockSpec((_NC, _BM, dn), lambda i: (0, i, 0)),
          pl.BlockSpec((_BM, _NC), lambda i: (i, 0)),
          pl.BlockSpec((_DH, dn), lambda i: (0, 0)),
          pl.BlockSpec((1, dn), lambda i: (0, 0)),
      ],
      out_specs=pl.BlockSpec((_BM, dn), lambda i: (i, 0)),
      out_shape=jax.ShapeDtypeStruct((_N, dn), jnp.float32),
  )(h, parts, cntT, Wr, b)


def kernel(x, edge_index, Wl1, Wr1, b1, g1, be1, Wl2, Wr2, b2, g2, be2,
           Wl3, Wr3, b3):
  src = edge_index[0]
  dst = edge_index[1]
  pad = _E_PAD - src.shape[0]
  ar = jnp.arange(pad, dtype=jnp.int32)
  src_p = jnp.concatenate([src, ar % _N])
  dst_p = jnp.concatenate([dst, _N + (ar % 16)])
  z2 = jnp.zeros((_N_ACC, _DH), jnp.float32)
  z1 = jnp.zeros((_N_ACC,), jnp.float32)

  seg_cnt = _make_seg_sum(_DH, True)
  seg128 = _make_seg_sum(_DH, False)

  b1r, g1r, be1r = b1.reshape(1, -1), g1.reshape(1, -1), be1.reshape(1, -1)
  b2r, g2r, be2r = b2.reshape(1, -1), g2.reshape(1, -1), be2.reshape(1, -1)
  b3r = b3.reshape(1, -1)

  p1 = _matmul(x, Wl1)
  parts1, cnt = seg_cnt(p1, src_p, dst_p, z2, z1)
  cntT = jnp.transpose(cnt.reshape(_NC, _N_ACC))
  h1, p2 = _stage(x, parts1, cntT, Wr1, b1r, g1r, be1r, Wl2)
  (parts2,) = seg128(p2, src_p, dst_p, z2, z1)
  h2, p3 = _stage(h1, parts2, cntT, Wr2, b2r, g2r, be2r, Wl3)
  seg2 = _make_seg_sum2()
  parts3 = seg2(p3.reshape(2 * _N), src_p, dst_p).reshape(_NC, _N_ACC, 2)
  return _final(h2, parts3, cntT, Wr3, b3r)
